# Initial kernel scaffold; baseline (speedup 1.0000x reference)
import jax
import jax.numpy as jnp
from jax.experimental import pallas as pl


def kernel(box_prediction, class_prediction):
    raise NotImplementedError("write your pallas kernel here")


# speedup vs baseline: 1.8214x; 1.8214x over previous
import functools

import jax
import jax.numpy as jnp
from jax import lax
from jax.experimental import pallas as pl
from jax.experimental.pallas import tpu as pltpu
from jax.experimental.pallas import tpu_sc as plsc

B, N, C = 4, 20000, 80
MAX_DET = 100
IOU_THR = 0.5
SCORE_THR = 0.5
NEG = -1e30

ROWS, LANES = 160, 128
NPAD = ROWS * LANES
NB = 10
NBLK = N // NB
DET_PAD = 128


def _softmax_body(x_ref, cls_ref, tab_ref, sc_ref):
    x = x_ref[0]
    t = x * 10.0
    sq = t * t
    m = jnp.max(sq, axis=-1, keepdims=True)
    e = jnp.exp(sq - m)
    s = jnp.sum(e, axis=-1, keepdims=True)
    p = e / s
    cls_ref[0] = p
    tab_ref[0, :, :C] = p
    tab_ref[0, :, C:] = jnp.zeros((NBLK, 128 - C), jnp.float32)
    score = 1.0 / s
    score_w = jnp.where(score >= SCORE_THR, score, NEG)
    sc_ref[0] = score_w.reshape(1, NBLK)


def _softmax_call(class_prediction):
    return pl.pallas_call(
        _softmax_body,
        grid=(B * NB,),
        in_specs=[pl.BlockSpec((1, NBLK, C), lambda i: (i // NB, i % NB, 0))],
        out_specs=[
            pl.BlockSpec((1, NBLK, C), lambda i: (i // NB, i % NB, 0)),
            pl.BlockSpec((1, NBLK, 128), lambda i: (i // NB, i % NB, 0)),
            pl.BlockSpec((1, 1, NBLK), lambda i: (i, 0, 0)),
        ],
        out_shape=[
            jax.ShapeDtypeStruct((B, N, C), jnp.float32),
            jax.ShapeDtypeStruct((B, N, 128), jnp.float32),
            jax.ShapeDtypeStruct((B * NB, 1, NBLK), jnp.float32),
        ],
    )(class_prediction)


def _nms_body(planes_ref, raw_ref, sc_ref, box_out, idx_out, val_out,
              s_ref, ymin_r, xmin_r, ymax_r, xmax_r, area_r, iota_r):
    b0 = planes_ref[0, 0]
    b1 = planes_ref[0, 1]
    b2 = planes_ref[0, 2]
    b3 = planes_ref[0, 3]
    ymin_r[...] = jnp.minimum(b0, b2)
    xmin_r[...] = jnp.minimum(b1, b3)
    ymax_r[...] = jnp.maximum(b0, b2)
    xmax_r[...] = jnp.maximum(b1, b3)
    area_r[...] = (ymax_r[...] - ymin_r[...]) * (xmax_r[...] - xmin_r[...])
    rr = lax.broadcasted_iota(jnp.int32, (ROWS, LANES), 0)
    cc = lax.broadcasted_iota(jnp.int32, (ROWS, LANES), 1)
    iota_r[...] = rr * LANES + cc
    s_ref[...] = sc_ref[0]

    def body(i, carry):
        s = s_ref[...]
        m = jnp.max(s)
        valid = m > NEG * 0.5
        idx2d = iota_r[...]
        best = jnp.min(jnp.where(s == m, idx2d, jnp.int32(2**31 - 1)))
        raw = raw_ref[0, pl.ds(best, 1), :]
        v0 = raw[0, 0]
        v1 = raw[0, 1]
        v2 = raw[0, 2]
        v3 = raw[0, 3]
        ybmin = jnp.minimum(v0, v2)
        xbmin = jnp.minimum(v1, v3)
        ybmax = jnp.maximum(v0, v2)
        xbmax = jnp.maximum(v1, v3)
        area_b = (ybmax - ybmin) * (xbmax - xbmin)
        ih = jnp.maximum(0.0, jnp.minimum(ybmax, ymax_r[...]) - jnp.maximum(ybmin, ymin_r[...]))
        iw = jnp.maximum(0.0, jnp.minimum(xbmax, xmax_r[...]) - jnp.maximum(xbmin, xmin_r[...]))
        inter = ih * iw
        union = area_b + area_r[...] - inter
        iou = jnp.where(union > 0.0, inter / union, 0.0)
        sup = (iou > IOU_THR) | (idx2d == best)
        s_ref[...] = jnp.where(jnp.logical_and(valid, sup), NEG, s)
        vf = jnp.where(valid, 1.0, 0.0)
        box_out[0, pl.ds(i, 1), :] = raw * vf
        idx_out[0, pl.ds(i, 1), :] = jnp.where(valid, best, 0).reshape(1, 1)
        val_out[0, pl.ds(i, 1), :] = vf.reshape(1, 1)
        return carry

    lax.fori_loop(0, MAX_DET, body, 0)


def _nms_call(planes, raw, scores):
    return pl.pallas_call(
        _nms_body,
        grid=(B,),
        in_specs=[
            pl.BlockSpec((1, 4, ROWS, LANES), lambda b: (b, 0, 0, 0)),
            pl.BlockSpec((1, NPAD, 4), lambda b: (b, 0, 0)),
            pl.BlockSpec((1, ROWS, LANES), lambda b: (b, 0, 0)),
        ],
        out_specs=[
            pl.BlockSpec((1, MAX_DET, 4), lambda b: (b, 0, 0)),
            pl.BlockSpec((1, MAX_DET, 1), lambda b: (b, 0, 0)),
            pl.BlockSpec((1, MAX_DET, 1), lambda b: (b, 0, 0)),
        ],
        out_shape=[
            jax.ShapeDtypeStruct((B, MAX_DET, 4), jnp.float32),
            jax.ShapeDtypeStruct((B, MAX_DET, 1), jnp.int32),
            jax.ShapeDtypeStruct((B, MAX_DET, 1), jnp.float32),
        ],
        scratch_shapes=[
            pltpu.VMEM((ROWS, LANES), jnp.float32),
            pltpu.VMEM((ROWS, LANES), jnp.float32),
            pltpu.VMEM((ROWS, LANES), jnp.float32),
            pltpu.VMEM((ROWS, LANES), jnp.float32),
            pltpu.VMEM((ROWS, LANES), jnp.float32),
            pltpu.VMEM((ROWS, LANES), jnp.float32),
            pltpu.VMEM((ROWS, LANES), jnp.int32),
        ],
    )(planes, raw, scores)


def _make_sc_gather():
    nc, ns, L = 2, 16, 16
    nw = nc * ns
    total = B * DET_PAD
    rpw = total // nw
    wpb = DET_PAD // rpw
    mesh = plsc.VectorSubcoreMesh(core_axis_name="c", subcore_axis_name="s")

    @functools.partial(
        pl.kernel,
        out_type=jax.ShapeDtypeStruct((total, 128), jnp.float32),
        mesh=mesh,
        scratch_types=[
            pltpu.VMEM((rpw,), jnp.int32),
            pltpu.VMEM((rpw, 128), jnp.float32),
            pltpu.SemaphoreType.DMA,
        ],
    )
    def sc_gather(table_hbm, idx_hbm, out_hbm, idx_v, rows_v, sem):
        wid = lax.axis_index("s") * nc + lax.axis_index("c")
        base = wid * rpw
        pltpu.sync_copy(idx_hbm.at[pl.ds(base, rpw)], idx_v)
        bi = wid // wpb
        idx_v[...] = idx_v[...] + bi * N
        pltpu.async_copy(table_hbm.at[idx_v], rows_v, sem).wait()
        pltpu.sync_copy(rows_v, out_hbm.at[pl.ds(base, rpw)])

    return sc_gather


def _mask_body(g_ref, v_ref, o_ref):
    o_ref[...] = g_ref[:, :C] * v_ref[...]


def _mask_call(g, val):
    total = B * DET_PAD
    return pl.pallas_call(
        _mask_body,
        out_shape=jax.ShapeDtypeStruct((total, C), jnp.float32),
    )(g, val)


_sc_gather_cache = []


def _get_sc_gather():
    if not _sc_gather_cache:
        _sc_gather_cache.append(_make_sc_gather())
    return _sc_gather_cache[0]


def kernel(box_prediction, class_prediction):
    cls_pred, table, scores3 = _softmax_call(class_prediction)
    scores = scores3.reshape(B, N)
    scores_p = jnp.pad(scores, ((0, 0), (0, NPAD - N)),
                       constant_values=NEG).reshape(B, ROWS, LANES)
    planes = jnp.transpose(box_prediction, (0, 2, 1))
    planes = jnp.pad(planes, ((0, 0), (0, 0), (0, NPAD - N))).reshape(B, 4, ROWS, LANES)
    raw = jnp.pad(box_prediction, ((0, 0), (0, NPAD - N), (0, 0)))
    nms_box, sel_idx, sel_val = _nms_call(planes, raw, scores_p)
    idx_flat = jnp.pad(sel_idx[:, :, 0], ((0, 0), (0, DET_PAD - MAX_DET))).reshape(-1)
    val_flat = jnp.pad(sel_val[:, :, 0], ((0, 0), (0, DET_PAD - MAX_DET))).reshape(-1, 1)
    g = _get_sc_gather()(table.reshape(B * N, 128), idx_flat)
    g = _mask_call(g, val_flat)
    nms_cls = g.reshape(B, DET_PAD, C)[:, :MAX_DET, :]
    return nms_box, nms_cls, cls_pred


# speedup vs baseline: 1.9799x; 1.0870x over previous
import functools

import jax
import jax.numpy as jnp
from jax import lax
from jax.experimental import pallas as pl
from jax.experimental.pallas import tpu as pltpu
from jax.experimental.pallas import tpu_sc as plsc

B, N, C = 4, 20000, 80
MAX_DET = 100
IOU_THR = 0.5
SCORE_THR = 0.5
NEG = -1e30

ROWS, LANES = 160, 128
NPAD = ROWS * LANES
NB = 10
NBLK = N // NB
DET_PAD = 128


def _softmax_body(x_ref, cls_ref, tab_ref, sc_ref):
    x = x_ref[0]
    t = x * 10.0
    sq = t * t
    m = jnp.max(sq, axis=-1, keepdims=True)
    e = jnp.exp(sq - m)
    s = jnp.sum(e, axis=-1, keepdims=True)
    p = e / s
    cls_ref[0] = p
    tab_ref[0, :, :C] = p
    tab_ref[0, :, C:] = jnp.zeros((NBLK, 128 - C), jnp.float32)
    score = 1.0 / s
    score_w = jnp.where(score >= SCORE_THR, score, NEG)
    sc_ref[0] = score_w.reshape(1, NBLK)


def _softmax_call(class_prediction):
    return pl.pallas_call(
        _softmax_body,
        grid=(B * NB,),
        in_specs=[pl.BlockSpec((1, NBLK, C), lambda i: (i // NB, i % NB, 0))],
        out_specs=[
            pl.BlockSpec((1, NBLK, C), lambda i: (i // NB, i % NB, 0)),
            pl.BlockSpec((1, NBLK, 128), lambda i: (i // NB, i % NB, 0)),
            pl.BlockSpec((1, 1, NBLK), lambda i: (i, 0, 0)),
        ],
        out_shape=[
            jax.ShapeDtypeStruct((B, N, C), jnp.float32),
            jax.ShapeDtypeStruct((B, N, 128), jnp.float32),
            jax.ShapeDtypeStruct((B * NB, 1, NBLK), jnp.float32),
        ],
    )(class_prediction)


def _nms_body(planes_ref, raw_ref, sc_ref, box_out, idx_out, val_out,
              s_ref, ymin_r, xmin_r, ymax_r, xmax_r, area_r, iota_r):
    for b in range(B):
        b0 = planes_ref[b, 0]
        b1 = planes_ref[b, 1]
        b2 = planes_ref[b, 2]
        b3 = planes_ref[b, 3]
        ymin_r[b] = jnp.minimum(b0, b2)
        xmin_r[b] = jnp.minimum(b1, b3)
        ymax_r[b] = jnp.maximum(b0, b2)
        xmax_r[b] = jnp.maximum(b1, b3)
        area_r[b] = (ymax_r[b] - ymin_r[b]) * (xmax_r[b] - xmin_r[b])
    rr = lax.broadcasted_iota(jnp.int32, (ROWS, LANES), 0)
    cc = lax.broadcasted_iota(jnp.int32, (ROWS, LANES), 1)
    iota_r[...] = rr * LANES + cc
    s_ref[...] = sc_ref[...]

    def body(i, carry):
        for b in range(B):
            s = s_ref[b]
            m = jnp.max(s)
            valid = m > NEG * 0.5
            idx2d = iota_r[...]
            best = jnp.min(jnp.where(s == m, idx2d, jnp.int32(2**31 - 1)))
            raw = raw_ref[b, pl.ds(best, 1), :]
            v0 = raw[0, 0]
            v1 = raw[0, 1]
            v2 = raw[0, 2]
            v3 = raw[0, 3]
            ybmin = jnp.minimum(v0, v2)
            xbmin = jnp.minimum(v1, v3)
            ybmax = jnp.maximum(v0, v2)
            xbmax = jnp.maximum(v1, v3)
            area_b = (ybmax - ybmin) * (xbmax - xbmin)
            ih = jnp.maximum(0.0, jnp.minimum(ybmax, ymax_r[b]) - jnp.maximum(ybmin, ymin_r[b]))
            iw = jnp.maximum(0.0, jnp.minimum(xbmax, xmax_r[b]) - jnp.maximum(xbmin, xmin_r[b]))
            inter = ih * iw
            union = area_b + area_r[b] - inter
            iou = jnp.where(union > 0.0, inter / union, 0.0)
            sup = (iou > IOU_THR) | (idx2d == best)
            s_ref[b] = jnp.where(jnp.logical_and(valid, sup), NEG, s)
            vf = jnp.where(valid, 1.0, 0.0)
            box_out[b, pl.ds(i, 1), :] = raw * vf
            idx_out[b, pl.ds(i, 1), :] = jnp.where(valid, best, 0).reshape(1, 1)
            val_out[b, pl.ds(i, 1), :] = vf.reshape(1, 1)
        return carry

    lax.fori_loop(0, MAX_DET, body, 0)


def _nms_call(planes, raw, scores):
    return pl.pallas_call(
        _nms_body,
        out_shape=[
            jax.ShapeDtypeStruct((B, MAX_DET, 4), jnp.float32),
            jax.ShapeDtypeStruct((B, MAX_DET, 1), jnp.int32),
            jax.ShapeDtypeStruct((B, MAX_DET, 1), jnp.float32),
        ],
        scratch_shapes=[
            pltpu.VMEM((B, ROWS, LANES), jnp.float32),
            pltpu.VMEM((B, ROWS, LANES), jnp.float32),
            pltpu.VMEM((B, ROWS, LANES), jnp.float32),
            pltpu.VMEM((B, ROWS, LANES), jnp.float32),
            pltpu.VMEM((B, ROWS, LANES), jnp.float32),
            pltpu.VMEM((B, ROWS, LANES), jnp.float32),
            pltpu.VMEM((ROWS, LANES), jnp.int32),
        ],
    )(planes, raw, scores)


def _make_sc_gather():
    nc, ns, L = 2, 16, 16
    nw = nc * ns
    total = B * DET_PAD
    rpw = total // nw
    wpb = DET_PAD // rpw
    mesh = plsc.VectorSubcoreMesh(core_axis_name="c", subcore_axis_name="s")

    @functools.partial(
        pl.kernel,
        out_type=jax.ShapeDtypeStruct((total, 128), jnp.float32),
        mesh=mesh,
        scratch_types=[
            pltpu.VMEM((rpw,), jnp.int32),
            pltpu.VMEM((rpw, 128), jnp.float32),
            pltpu.SemaphoreType.DMA,
        ],
    )
    def sc_gather(table_hbm, idx_hbm, out_hbm, idx_v, rows_v, sem):
        wid = lax.axis_index("s") * nc + lax.axis_index("c")
        base = wid * rpw
        pltpu.sync_copy(idx_hbm.at[pl.ds(base, rpw)], idx_v)
        bi = wid // wpb
        idx_v[...] = idx_v[...] + bi * N
        pltpu.async_copy(table_hbm.at[idx_v], rows_v, sem).wait()
        pltpu.sync_copy(rows_v, out_hbm.at[pl.ds(base, rpw)])

    return sc_gather


def _mask_body(g_ref, v_ref, o_ref):
    o_ref[...] = g_ref[:, :C] * v_ref[...]


def _mask_call(g, val):
    total = B * DET_PAD
    return pl.pallas_call(
        _mask_body,
        out_shape=jax.ShapeDtypeStruct((total, C), jnp.float32),
    )(g, val)


_sc_gather_cache = []


def _get_sc_gather():
    if not _sc_gather_cache:
        _sc_gather_cache.append(_make_sc_gather())
    return _sc_gather_cache[0]


def kernel(box_prediction, class_prediction):
    cls_pred, table, scores3 = _softmax_call(class_prediction)
    scores = scores3.reshape(B, N)
    scores_p = jnp.pad(scores, ((0, 0), (0, NPAD - N)),
                       constant_values=NEG).reshape(B, ROWS, LANES)
    planes = jnp.transpose(box_prediction, (0, 2, 1))
    planes = jnp.pad(planes, ((0, 0), (0, 0), (0, NPAD - N))).reshape(B, 4, ROWS, LANES)
    raw = jnp.pad(box_prediction, ((0, 0), (0, NPAD - N), (0, 0)))
    nms_box, sel_idx, sel_val = _nms_call(planes, raw, scores_p)
    idx_flat = jnp.pad(sel_idx[:, :, 0], ((0, 0), (0, DET_PAD - MAX_DET))).reshape(-1)
    val_flat = jnp.pad(sel_val[:, :, 0], ((0, 0), (0, DET_PAD - MAX_DET))).reshape(-1, 1)
    g = _get_sc_gather()(table.reshape(B * N, 128), idx_flat)
    g = _mask_call(g, val_flat)
    nms_cls = g.reshape(B, DET_PAD, C)[:, :MAX_DET, :]
    return nms_box, nms_cls, cls_pred


# speedup vs baseline: 2.0932x; 1.0572x over previous
import functools

import jax
import jax.numpy as jnp
from jax import lax
from jax.experimental import pallas as pl
from jax.experimental.pallas import tpu as pltpu
from jax.experimental.pallas import tpu_sc as plsc

B, N, C = 4, 20000, 80
MAX_DET = 100
IOU_THR = 0.5
SCORE_THR = 0.5
NEG = -1e30

ROWS, LANES = 160, 128
NPAD = ROWS * LANES
NB = 10
NBLK = N // NB
DET_PAD = 128


def _softmax_body(x_ref, cls_ref, tab_ref, sc_ref):
    x = x_ref[0]
    t = x * 10.0
    sq = t * t
    m = jnp.max(sq, axis=-1, keepdims=True)
    e = jnp.exp(sq - m)
    s = jnp.sum(e, axis=-1, keepdims=True)
    p = e / s
    cls_ref[0] = p
    tab_ref[0, :, :C] = p
    tab_ref[0, :, C:] = jnp.zeros((NBLK, 128 - C), jnp.float32)
    score = 1.0 / s
    score_w = jnp.where(score >= SCORE_THR, score, NEG)
    sc_ref[0] = score_w.reshape(1, NBLK)


def _softmax_call(class_prediction):
    return pl.pallas_call(
        _softmax_body,
        grid=(B * NB,),
        in_specs=[pl.BlockSpec((1, NBLK, C), lambda i: (i // NB, i % NB, 0))],
        out_specs=[
            pl.BlockSpec((1, NBLK, C), lambda i: (i // NB, i % NB, 0)),
            pl.BlockSpec((1, NBLK, 128), lambda i: (i // NB, i % NB, 0)),
            pl.BlockSpec((1, 1, NBLK), lambda i: (i, 0, 0)),
        ],
        out_shape=[
            jax.ShapeDtypeStruct((B, N, C), jnp.float32),
            jax.ShapeDtypeStruct((B, N, 128), jnp.float32),
            jax.ShapeDtypeStruct((B * NB, 1, NBLK), jnp.float32),
        ],
    )(class_prediction)


def _nms_body(planes_ref, raw_ref, sc_ref, box_out, idx_out, val_out,
              s0, s1, s2, s3, ymin_r, xmin_r, ymax_r, xmax_r, area_r):
    s_refs = (s0, s1, s2, s3)
    for b in range(B):
        b0 = planes_ref[b, 0]
        b1 = planes_ref[b, 1]
        b2 = planes_ref[b, 2]
        b3 = planes_ref[b, 3]
        ymin_r[b] = jnp.minimum(b0, b2)
        xmin_r[b] = jnp.minimum(b1, b3)
        ymax_r[b] = jnp.maximum(b0, b2)
        xmax_r[b] = jnp.maximum(b1, b3)
        area_r[b] = (ymax_r[b] - ymin_r[b]) * (xmax_r[b] - xmin_r[b])
        s_refs[b][...] = sc_ref[b]
    m_init = tuple(jnp.max(sc_ref[b]) for b in range(B))

    def body(i, carry):
        rr = lax.broadcasted_iota(jnp.int32, (ROWS, LANES), 0)
        cc = lax.broadcasted_iota(jnp.int32, (ROWS, LANES), 1)
        idx2d = rr * LANES + cc
        new_ms = []
        for b in range(B):
            m = carry[b]
            s = s_refs[b][...]
            valid = m > NEG * 0.5
            best = jnp.min(jnp.where(s == m, idx2d, jnp.int32(2**31 - 1)))
            raw = raw_ref[b, pl.ds(best, 1), :]
            v0 = raw[0, 0]
            v1 = raw[0, 1]
            v2 = raw[0, 2]
            v3 = raw[0, 3]
            ybmin = jnp.minimum(v0, v2)
            xbmin = jnp.minimum(v1, v3)
            ybmax = jnp.maximum(v0, v2)
            xbmax = jnp.maximum(v1, v3)
            area_b = (ybmax - ybmin) * (xbmax - xbmin)
            ih = jnp.maximum(0.0, jnp.minimum(ybmax, ymax_r[b]) - jnp.maximum(ybmin, ymin_r[b]))
            iw = jnp.maximum(0.0, jnp.minimum(xbmax, xmax_r[b]) - jnp.maximum(xbmin, xmin_r[b]))
            inter = ih * iw
            union = area_b + area_r[b] - inter
            iou = jnp.where(union > 0.0, inter / union, 0.0)
            sup = (iou > IOU_THR) | (idx2d == best)
            new_s = jnp.where(jnp.logical_and(valid, sup), NEG, s)
            s_refs[b][...] = new_s
            new_ms.append(jnp.max(new_s))
            vf = jnp.where(valid, 1.0, 0.0)
            box_out[b, pl.ds(i, 1), :] = raw * vf
            idx_out[b, pl.ds(i, 1), :] = jnp.where(valid, best, 0).reshape(1, 1)
            val_out[b, pl.ds(i, 1), :] = vf.reshape(1, 1)
        return tuple(new_ms)

    lax.fori_loop(0, MAX_DET, body, m_init)


def _nms_call(planes, raw, scores):
    return pl.pallas_call(
        _nms_body,
        out_shape=[
            jax.ShapeDtypeStruct((B, MAX_DET, 4), jnp.float32),
            jax.ShapeDtypeStruct((B, MAX_DET, 1), jnp.int32),
            jax.ShapeDtypeStruct((B, MAX_DET, 1), jnp.float32),
        ],
        scratch_shapes=[
            pltpu.VMEM((ROWS, LANES), jnp.float32),
            pltpu.VMEM((ROWS, LANES), jnp.float32),
            pltpu.VMEM((ROWS, LANES), jnp.float32),
            pltpu.VMEM((ROWS, LANES), jnp.float32),
            pltpu.VMEM((B, ROWS, LANES), jnp.float32),
            pltpu.VMEM((B, ROWS, LANES), jnp.float32),
            pltpu.VMEM((B, ROWS, LANES), jnp.float32),
            pltpu.VMEM((B, ROWS, LANES), jnp.float32),
            pltpu.VMEM((B, ROWS, LANES), jnp.float32),
        ],
    )(planes, raw, scores)


def _make_sc_gather():
    nc, ns, L = 2, 16, 16
    nw = nc * ns
    total = B * DET_PAD
    rpw = total // nw
    wpb = DET_PAD // rpw
    mesh = plsc.VectorSubcoreMesh(core_axis_name="c", subcore_axis_name="s")

    @functools.partial(
        pl.kernel,
        out_type=jax.ShapeDtypeStruct((total, 128), jnp.float32),
        mesh=mesh,
        scratch_types=[
            pltpu.VMEM((rpw,), jnp.int32),
            pltpu.VMEM((rpw, 128), jnp.float32),
            pltpu.SemaphoreType.DMA,
        ],
    )
    def sc_gather(table_hbm, idx_hbm, out_hbm, idx_v, rows_v, sem):
        wid = lax.axis_index("s") * nc + lax.axis_index("c")
        base = wid * rpw
        pltpu.sync_copy(idx_hbm.at[pl.ds(base, rpw)], idx_v)
        bi = wid // wpb
        idx_v[...] = idx_v[...] + bi * N
        pltpu.async_copy(table_hbm.at[idx_v], rows_v, sem).wait()
        pltpu.sync_copy(rows_v, out_hbm.at[pl.ds(base, rpw)])

    return sc_gather


def _mask_body(g_ref, v_ref, o_ref):
    o_ref[...] = g_ref[:, :C] * v_ref[...]


def _mask_call(g, val):
    total = B * DET_PAD
    return pl.pallas_call(
        _mask_body,
        out_shape=jax.ShapeDtypeStruct((total, C), jnp.float32),
    )(g, val)


_sc_gather_cache = []


def _get_sc_gather():
    if not _sc_gather_cache:
        _sc_gather_cache.append(_make_sc_gather())
    return _sc_gather_cache[0]


def kernel(box_prediction, class_prediction):
    cls_pred, table, scores3 = _softmax_call(class_prediction)
    scores = scores3.reshape(B, N)
    scores_p = jnp.pad(scores, ((0, 0), (0, NPAD - N)),
                       constant_values=NEG).reshape(B, ROWS, LANES)
    planes = jnp.transpose(box_prediction, (0, 2, 1))
    planes = jnp.pad(planes, ((0, 0), (0, 0), (0, NPAD - N))).reshape(B, 4, ROWS, LANES)
    raw = jnp.pad(box_prediction, ((0, 0), (0, NPAD - N), (0, 0)))
    nms_box, sel_idx, sel_val = _nms_call(planes, raw, scores_p)
    idx_flat = jnp.pad(sel_idx[:, :, 0], ((0, 0), (0, DET_PAD - MAX_DET))).reshape(-1)
    val_flat = jnp.pad(sel_val[:, :, 0], ((0, 0), (0, DET_PAD - MAX_DET))).reshape(-1, 1)
    g = _get_sc_gather()(table.reshape(B * N, 128), idx_flat)
    g = _mask_call(g, val_flat)
    nms_cls = g.reshape(B, DET_PAD, C)[:, :MAX_DET, :]
    return nms_box, nms_cls, cls_pred


# speedup vs baseline: 2.5757x; 1.2305x over previous
import functools

import jax
import jax.numpy as jnp
from jax import lax
from jax.experimental import pallas as pl
from jax.experimental.pallas import tpu as pltpu
from jax.experimental.pallas import tpu_sc as plsc

B, N, C = 4, 20000, 80
MAX_DET = 100
IOU_THR = 0.5
SCORE_THR = 0.5
NEG = -1e30

ROWS, LANES = 160, 128
NPAD = ROWS * LANES
NB = 10
NBLK = N // NB
DET_PAD = 128


def _softmax_body(x_ref, cls_ref, tab_ref, sc_ref):
    x = x_ref[0]
    t = x * 10.0
    sq = t * t
    m = jnp.max(sq, axis=-1, keepdims=True)
    e = jnp.exp(sq - m)
    s = jnp.sum(e, axis=-1, keepdims=True)
    p = e / s
    cls_ref[0] = p
    tab_ref[0, :, :C] = p
    tab_ref[0, :, C:] = jnp.zeros((NBLK, 128 - C), jnp.float32)
    score = 1.0 / s
    score_w = jnp.where(score >= SCORE_THR, score, NEG)
    sc_ref[0] = score_w.reshape(1, NBLK)


def _softmax_call(class_prediction):
    return pl.pallas_call(
        _softmax_body,
        grid=(B * NB,),
        in_specs=[pl.BlockSpec((1, NBLK, C), lambda i: (i // NB, i % NB, 0))],
        out_specs=[
            pl.BlockSpec((1, NBLK, C), lambda i: (i // NB, i % NB, 0)),
            pl.BlockSpec((1, NBLK, 128), lambda i: (i // NB, i % NB, 0)),
            pl.BlockSpec((1, 1, NBLK), lambda i: (i, 0, 0)),
        ],
        out_shape=[
            jax.ShapeDtypeStruct((B, N, C), jnp.float32),
            jax.ShapeDtypeStruct((B, N, 128), jnp.float32),
            jax.ShapeDtypeStruct((B * NB, 1, NBLK), jnp.float32),
        ],
    )(class_prediction)


def _nms_body(planes_ref, raw_ref, sc_ref, box_out, idx_out, val_out,
              s0, s1, s2, s3, ymin_r, xmin_r, ymax_r, xmax_r, area_r):
    s_refs = (s0, s1, s2, s3)
    for b in range(B):
        b0 = planes_ref[b, 0]
        b1 = planes_ref[b, 1]
        b2 = planes_ref[b, 2]
        b3 = planes_ref[b, 3]
        ymin_r[b] = jnp.minimum(b0, b2)
        xmin_r[b] = jnp.minimum(b1, b3)
        ymax_r[b] = jnp.maximum(b0, b2)
        xmax_r[b] = jnp.maximum(b1, b3)
        area_r[b] = (ymax_r[b] - ymin_r[b]) * (xmax_r[b] - xmin_r[b])
        s_refs[b][...] = sc_ref[b]
    m_init = tuple(jnp.max(sc_ref[b]) for b in range(B))

    def body(i, carry):
        rr = lax.broadcasted_iota(jnp.int32, (ROWS, LANES), 0)
        cc = lax.broadcasted_iota(jnp.int32, (ROWS, LANES), 1)
        idx2d = rr * LANES + cc
        svals = [s_refs[b][...] for b in range(B)]
        valids = [carry[b] > NEG * 0.5 for b in range(B)]
        bests = [jnp.min(jnp.where(svals[b] == carry[b], idx2d, jnp.int32(2**31 - 1)))
                 for b in range(B)]
        raws = [raw_ref[b, pl.ds(bests[b], 1), :] for b in range(B)]
        new_ms = []
        sups = []
        for b in range(B):
            raw = raws[b]
            v0 = raw[0, 0]
            v1 = raw[0, 1]
            v2 = raw[0, 2]
            v3 = raw[0, 3]
            ybmin = jnp.minimum(v0, v2)
            xbmin = jnp.minimum(v1, v3)
            ybmax = jnp.maximum(v0, v2)
            xbmax = jnp.maximum(v1, v3)
            area_b = (ybmax - ybmin) * (xbmax - xbmin)
            ih = jnp.maximum(0.0, jnp.minimum(ybmax, ymax_r[b]) - jnp.maximum(ybmin, ymin_r[b]))
            iw = jnp.maximum(0.0, jnp.minimum(xbmax, xmax_r[b]) - jnp.maximum(xbmin, xmin_r[b]))
            inter = ih * iw
            union = area_b + area_r[b] - inter
            iou = jnp.where(union > 0.0, inter / union, 0.0)
            sups.append((iou > IOU_THR) | (idx2d == bests[b]))
        for b in range(B):
            new_s = jnp.where(jnp.logical_and(valids[b], sups[b]), NEG, svals[b])
            s_refs[b][...] = new_s
            new_ms.append(jnp.max(new_s))
        for b in range(B):
            vf = jnp.where(valids[b], 1.0, 0.0)
            box_out[b, pl.ds(i, 1), :] = raws[b] * vf
            idx_out[b, pl.ds(i, 1), :] = jnp.where(valids[b], bests[b], 0).reshape(1, 1)
            val_out[b, pl.ds(i, 1), :] = vf.reshape(1, 1)
        return tuple(new_ms)

    lax.fori_loop(0, MAX_DET, body, m_init)


def _nms_call(planes, raw, scores):
    return pl.pallas_call(
        _nms_body,
        out_shape=[
            jax.ShapeDtypeStruct((B, MAX_DET, 4), jnp.float32),
            jax.ShapeDtypeStruct((B, MAX_DET, 1), jnp.int32),
            jax.ShapeDtypeStruct((B, MAX_DET, 1), jnp.float32),
        ],
        scratch_shapes=[
            pltpu.VMEM((ROWS, LANES), jnp.float32),
            pltpu.VMEM((ROWS, LANES), jnp.float32),
            pltpu.VMEM((ROWS, LANES), jnp.float32),
            pltpu.VMEM((ROWS, LANES), jnp.float32),
            pltpu.VMEM((B, ROWS, LANES), jnp.float32),
            pltpu.VMEM((B, ROWS, LANES), jnp.float32),
            pltpu.VMEM((B, ROWS, LANES), jnp.float32),
            pltpu.VMEM((B, ROWS, LANES), jnp.float32),
            pltpu.VMEM((B, ROWS, LANES), jnp.float32),
        ],
    )(planes, raw, scores)


def _make_sc_gather():
    nc, ns, L = 2, 16, 16
    nw = nc * ns
    total = B * DET_PAD
    rpw = total // nw
    wpb = DET_PAD // rpw
    mesh = plsc.VectorSubcoreMesh(core_axis_name="c", subcore_axis_name="s")

    @functools.partial(
        pl.kernel,
        out_type=jax.ShapeDtypeStruct((total, 128), jnp.float32),
        mesh=mesh,
        scratch_types=[
            pltpu.VMEM((rpw,), jnp.int32),
            pltpu.VMEM((rpw, 128), jnp.float32),
            pltpu.SemaphoreType.DMA,
        ],
    )
    def sc_gather(table_hbm, idx_hbm, out_hbm, idx_v, rows_v, sem):
        wid = lax.axis_index("s") * nc + lax.axis_index("c")
        base = wid * rpw
        pltpu.sync_copy(idx_hbm.at[pl.ds(base, rpw)], idx_v)
        bi = wid // wpb
        idx_v[...] = idx_v[...] + bi * N
        pltpu.async_copy(table_hbm.at[idx_v], rows_v, sem).wait()
        pltpu.sync_copy(rows_v, out_hbm.at[pl.ds(base, rpw)])

    return sc_gather


def _mask_body(g_ref, v_ref, o_ref):
    o_ref[...] = g_ref[:, :C] * v_ref[...]


def _mask_call(g, val):
    total = B * DET_PAD
    return pl.pallas_call(
        _mask_body,
        out_shape=jax.ShapeDtypeStruct((total, C), jnp.float32),
    )(g, val)


_sc_gather_cache = []


def _get_sc_gather():
    if not _sc_gather_cache:
        _sc_gather_cache.append(_make_sc_gather())
    return _sc_gather_cache[0]


def kernel(box_prediction, class_prediction):
    cls_pred, table, scores3 = _softmax_call(class_prediction)
    scores = scores3.reshape(B, N)
    scores_p = jnp.pad(scores, ((0, 0), (0, NPAD - N)),
                       constant_values=NEG).reshape(B, ROWS, LANES)
    planes = jnp.transpose(box_prediction, (0, 2, 1))
    planes = jnp.pad(planes, ((0, 0), (0, 0), (0, NPAD - N))).reshape(B, 4, ROWS, LANES)
    raw = jnp.pad(box_prediction, ((0, 0), (0, NPAD - N), (0, 0)))
    nms_box, sel_idx, sel_val = _nms_call(planes, raw, scores_p)
    idx_flat = jnp.pad(sel_idx[:, :, 0], ((0, 0), (0, DET_PAD - MAX_DET))).reshape(-1)
    val_flat = jnp.pad(sel_val[:, :, 0], ((0, 0), (0, DET_PAD - MAX_DET))).reshape(-1, 1)
    g = _get_sc_gather()(table.reshape(B * N, 128), idx_flat)
    g = _mask_call(g, val_flat)
    nms_cls = g.reshape(B, DET_PAD, C)[:, :MAX_DET, :]
    return nms_box, nms_cls, cls_pred


# speedup vs baseline: 2.8706x; 1.1145x over previous
import functools

import jax
import jax.numpy as jnp
from jax import lax
from jax.experimental import pallas as pl
from jax.experimental.pallas import tpu as pltpu
from jax.experimental.pallas import tpu_sc as plsc

B, N, C = 4, 20000, 80
MAX_DET = 100
IOU_THR = 0.5
SCORE_THR = 0.5
NEG = -1e30

ROWS, LANES = 160, 128
NPAD = ROWS * LANES
NB = 10
NBLK = N // NB
DET_PAD = 128


def _softmax_body(x_ref, cls_ref, tab_ref, sc_ref):
    x = x_ref[0]
    t = x * 10.0
    sq = t * t
    m = jnp.max(sq, axis=-1, keepdims=True)
    e = jnp.exp(sq - m)
    s = jnp.sum(e, axis=-1, keepdims=True)
    p = e / s
    cls_ref[0] = p
    tab_ref[0, :, :C] = p
    tab_ref[0, :, C:] = jnp.zeros((NBLK, 128 - C), jnp.float32)
    score = 1.0 / s
    score_w = jnp.where(score >= SCORE_THR, score, NEG)
    sc_ref[0] = score_w.reshape(1, NBLK)


def _softmax_call(class_prediction):
    return pl.pallas_call(
        _softmax_body,
        grid=(B * NB,),
        in_specs=[pl.BlockSpec((1, NBLK, C), lambda i: (i // NB, i % NB, 0))],
        out_specs=[
            pl.BlockSpec((1, NBLK, C), lambda i: (i // NB, i % NB, 0)),
            pl.BlockSpec((1, NBLK, 128), lambda i: (i // NB, i % NB, 0)),
            pl.BlockSpec((1, 1, NBLK), lambda i: (i, 0, 0)),
        ],
        out_shape=[
            jax.ShapeDtypeStruct((B, N, C), jnp.float32),
            jax.ShapeDtypeStruct((B, N, 128), jnp.float32),
            jax.ShapeDtypeStruct((B * NB, 1, NBLK), jnp.float32),
        ],
    )(class_prediction)


def _nms_body(planes_ref, raw_ref, sc_ref, box_out, idx_out, val_out,
              s0, s1, s2, s3, ymin_r, xmin_r, ymax_r, xmax_r, area_r):
    s_refs = (s0, s1, s2, s3)
    idx_out[...] = jnp.zeros((B, DET_PAD, 1), jnp.int32)
    val_out[...] = jnp.zeros((B, DET_PAD, 1), jnp.float32)
    for b in range(B):
        b0 = planes_ref[b, 0]
        b1 = planes_ref[b, 1]
        b2 = planes_ref[b, 2]
        b3 = planes_ref[b, 3]
        ymin_r[b] = jnp.minimum(b0, b2)
        xmin_r[b] = jnp.minimum(b1, b3)
        ymax_r[b] = jnp.maximum(b0, b2)
        xmax_r[b] = jnp.maximum(b1, b3)
        area_r[b] = (ymax_r[b] - ymin_r[b]) * (xmax_r[b] - xmin_r[b])
        s_refs[b][...] = sc_ref[b]
    m_init = tuple(jnp.max(sc_ref[b]) for b in range(B))

    def body(i, carry):
        rr = lax.broadcasted_iota(jnp.int32, (ROWS, LANES), 0)
        cc = lax.broadcasted_iota(jnp.int32, (ROWS, LANES), 1)
        idx2d = rr * LANES + cc
        svals = [s_refs[b][...] for b in range(B)]
        valids = [carry[b] > NEG * 0.5 for b in range(B)]
        bests = [jnp.min(jnp.where(svals[b] == carry[b], idx2d, jnp.int32(2**31 - 1)))
                 for b in range(B)]
        raws = [raw_ref[b, pl.ds(bests[b], 1), :] for b in range(B)]
        new_ms = []
        sups = []
        for b in range(B):
            raw = raws[b]
            v0 = raw[0, 0]
            v1 = raw[0, 1]
            v2 = raw[0, 2]
            v3 = raw[0, 3]
            ybmin = jnp.minimum(v0, v2)
            xbmin = jnp.minimum(v1, v3)
            ybmax = jnp.maximum(v0, v2)
            xbmax = jnp.maximum(v1, v3)
            area_b = (ybmax - ybmin) * (xbmax - xbmin)
            ih = jnp.maximum(0.0, jnp.minimum(ybmax, ymax_r[b]) - jnp.maximum(ybmin, ymin_r[b]))
            iw = jnp.maximum(0.0, jnp.minimum(xbmax, xmax_r[b]) - jnp.maximum(xbmin, xmin_r[b]))
            inter = ih * iw
            union = area_b + area_r[b] - inter
            iou = jnp.where(union > 0.0, inter / union, 0.0)
            sups.append((iou > IOU_THR) | (idx2d == bests[b]))
        for b in range(B):
            new_s = jnp.where(jnp.logical_and(valids[b], sups[b]), NEG, svals[b])
            s_refs[b][...] = new_s
            new_ms.append(jnp.max(new_s))
        for b in range(B):
            vf = jnp.where(valids[b], 1.0, 0.0)
            box_out[b, pl.ds(i, 1), :] = raws[b] * vf
            idx_out[b, pl.ds(i, 1), :] = jnp.where(valids[b], bests[b], 0).reshape(1, 1)
            val_out[b, pl.ds(i, 1), :] = vf.reshape(1, 1)
        return tuple(new_ms)

    lax.fori_loop(0, MAX_DET, body, m_init)


def _nms_call(planes, raw, scores):
    return pl.pallas_call(
        _nms_body,
        out_shape=[
            jax.ShapeDtypeStruct((B, MAX_DET, 4), jnp.float32),
            jax.ShapeDtypeStruct((B, DET_PAD, 1), jnp.int32),
            jax.ShapeDtypeStruct((B, DET_PAD, 1), jnp.float32),
        ],
        scratch_shapes=[
            pltpu.VMEM((ROWS, LANES), jnp.float32),
            pltpu.VMEM((ROWS, LANES), jnp.float32),
            pltpu.VMEM((ROWS, LANES), jnp.float32),
            pltpu.VMEM((ROWS, LANES), jnp.float32),
            pltpu.VMEM((B, ROWS, LANES), jnp.float32),
            pltpu.VMEM((B, ROWS, LANES), jnp.float32),
            pltpu.VMEM((B, ROWS, LANES), jnp.float32),
            pltpu.VMEM((B, ROWS, LANES), jnp.float32),
            pltpu.VMEM((B, ROWS, LANES), jnp.float32),
        ],
    )(planes, raw, scores)


def _make_sc_gather():
    nc, ns, L = 2, 16, 16
    nw = nc * ns
    total = B * DET_PAD
    rpw = total // nw
    wpb = DET_PAD // rpw
    mesh = plsc.VectorSubcoreMesh(core_axis_name="c", subcore_axis_name="s")

    @functools.partial(
        pl.kernel,
        out_type=jax.ShapeDtypeStruct((total, 128), jnp.float32),
        mesh=mesh,
        scratch_types=[
            pltpu.VMEM((rpw,), jnp.int32),
            pltpu.VMEM((rpw, 128), jnp.float32),
            pltpu.SemaphoreType.DMA,
        ],
    )
    def sc_gather(table_hbm, idx_hbm, out_hbm, idx_v, rows_v, sem):
        wid = lax.axis_index("s") * nc + lax.axis_index("c")
        base = wid * rpw
        pltpu.sync_copy(idx_hbm.at[pl.ds(base, rpw)], idx_v)
        bi = wid // wpb
        idx_v[...] = idx_v[...] + bi * N
        pltpu.async_copy(table_hbm.at[idx_v], rows_v, sem).wait()
        pltpu.sync_copy(rows_v, out_hbm.at[pl.ds(base, rpw)])

    return sc_gather


def _mask_body(g_ref, v_ref, o_ref):
    o_ref[0] = g_ref[0, :MAX_DET, :C] * v_ref[0, :MAX_DET]


def _mask_call(g, val):
    return pl.pallas_call(
        _mask_body,
        grid=(B,),
        in_specs=[
            pl.BlockSpec((1, DET_PAD, 128), lambda b: (b, 0, 0)),
            pl.BlockSpec((1, DET_PAD, 1), lambda b: (b, 0, 0)),
        ],
        out_specs=pl.BlockSpec((1, MAX_DET, C), lambda b: (b, 0, 0)),
        out_shape=jax.ShapeDtypeStruct((B, MAX_DET, C), jnp.float32),
    )(g, val)


_sc_gather_cache = []


def _get_sc_gather():
    if not _sc_gather_cache:
        _sc_gather_cache.append(_make_sc_gather())
    return _sc_gather_cache[0]


def kernel(box_prediction, class_prediction):
    cls_pred, table, scores3 = _softmax_call(class_prediction)
    scores = scores3.reshape(B, N)
    scores_p = jnp.pad(scores, ((0, 0), (0, NPAD - N)),
                       constant_values=NEG).reshape(B, ROWS, LANES)
    planes = jnp.transpose(box_prediction, (0, 2, 1))
    planes = jnp.pad(planes, ((0, 0), (0, 0), (0, NPAD - N))).reshape(B, 4, ROWS, LANES)
    nms_box, sel_idx, sel_val = _nms_call(planes, box_prediction, scores_p)
    g = _get_sc_gather()(table.reshape(B * N, 128), sel_idx.reshape(-1))
    nms_cls = _mask_call(g.reshape(B, DET_PAD, 128), sel_val)
    return nms_box, nms_cls, cls_pred


# speedup vs baseline: 2.9020x; 1.0109x over previous
import functools

import jax
import jax.numpy as jnp
from jax import lax
from jax.experimental import pallas as pl
from jax.experimental.pallas import tpu as pltpu
from jax.experimental.pallas import tpu_sc as plsc

B, N, C = 4, 20000, 80
MAX_DET = 100
IOU_THR = 0.5
SCORE_THR = 0.5
NEG = -1e30

ROWS, LANES = 160, 128
NPAD = ROWS * LANES
NB = 10
NBLK = N // NB
DET_PAD = 128


def _softmax_body(x_ref, cls_ref, tab_ref, sc_ref):
    x = x_ref[0]
    t = x * 10.0
    sq = t * t
    m = jnp.max(sq, axis=-1, keepdims=True)
    e = jnp.exp(sq - m)
    s = jnp.sum(e, axis=-1, keepdims=True)
    p = e / s
    cls_ref[0] = p
    tab_ref[0, :, :C] = p
    tab_ref[0, :, C:] = jnp.zeros((NBLK, 128 - C), jnp.float32)
    score = 1.0 / s
    score_w = jnp.where(score >= SCORE_THR, score, NEG)
    sc_ref[0] = score_w.reshape(1, NBLK)


def _softmax_call(class_prediction):
    return pl.pallas_call(
        _softmax_body,
        grid=(B * NB,),
        in_specs=[pl.BlockSpec((1, NBLK, C), lambda i: (i // NB, i % NB, 0))],
        out_specs=[
            pl.BlockSpec((1, NBLK, C), lambda i: (i // NB, i % NB, 0)),
            pl.BlockSpec((1, NBLK, 128), lambda i: (i // NB, i % NB, 0)),
            pl.BlockSpec((1, 1, NBLK), lambda i: (i, 0, 0)),
        ],
        out_shape=[
            jax.ShapeDtypeStruct((B, N, C), jnp.float32),
            jax.ShapeDtypeStruct((B, N, 128), jnp.float32),
            jax.ShapeDtypeStruct((B * NB, 1, NBLK), jnp.float32),
        ],
    )(class_prediction)


def _nms_body(planes_ref, raw_ref, sc_ref, box_out, idx_out, val_out,
              s0, s1, s2, s3, ymin_r, xmin_r, ymax_r, xmax_r, area_r):
    s_refs = (s0, s1, s2, s3)
    idx_out[...] = jnp.zeros((B, DET_PAD, 1), jnp.int32)
    val_out[...] = jnp.zeros((B, DET_PAD, 1), jnp.float32)
    for b in range(B):
        b0 = planes_ref[b, 0]
        b1 = planes_ref[b, 1]
        b2 = planes_ref[b, 2]
        b3 = planes_ref[b, 3]
        ymin_r[b] = jnp.minimum(b0, b2)
        xmin_r[b] = jnp.minimum(b1, b3)
        ymax_r[b] = jnp.maximum(b0, b2)
        xmax_r[b] = jnp.maximum(b1, b3)
        area_r[b] = (ymax_r[b] - ymin_r[b]) * (xmax_r[b] - xmin_r[b])
        s_refs[b][...] = sc_ref[b]
    m_init = tuple(jnp.max(sc_ref[b]) for b in range(B))

    def body(i, carry):
        rr = lax.broadcasted_iota(jnp.int32, (ROWS, LANES), 0)
        cc = lax.broadcasted_iota(jnp.int32, (ROWS, LANES), 1)
        idx2d = rr * LANES + cc
        svals = [s_refs[b][...] for b in range(B)]
        valids = [carry[b] > NEG * 0.5 for b in range(B)]
        bests = [jnp.min(jnp.where(svals[b] == carry[b], idx2d, jnp.int32(2**31 - 1)))
                 for b in range(B)]
        raws = [raw_ref[b, pl.ds(bests[b], 1), :] for b in range(B)]
        new_ms = []
        sups = []
        for b in range(B):
            raw = raws[b]
            v0 = raw[0, 0]
            v1 = raw[0, 1]
            v2 = raw[0, 2]
            v3 = raw[0, 3]
            ybmin = jnp.minimum(v0, v2)
            xbmin = jnp.minimum(v1, v3)
            ybmax = jnp.maximum(v0, v2)
            xbmax = jnp.maximum(v1, v3)
            area_b = (ybmax - ybmin) * (xbmax - xbmin)
            ih = jnp.maximum(0.0, jnp.minimum(ybmax, ymax_r[b]) - jnp.maximum(ybmin, ymin_r[b]))
            iw = jnp.maximum(0.0, jnp.minimum(xbmax, xmax_r[b]) - jnp.maximum(xbmin, xmin_r[b]))
            inter = ih * iw
            union = area_b + area_r[b] - inter
            iou = jnp.where(union > 0.0, inter / union, 0.0)
            sups.append((iou > IOU_THR) | (idx2d == bests[b]))
        for b in range(B):
            new_s = jnp.where(sups[b], NEG, svals[b])
            s_refs[b][...] = new_s
            new_ms.append(jnp.max(new_s))
        for b in range(B):
            vf = jnp.where(valids[b], 1.0, 0.0)
            box_out[b, pl.ds(i, 1), :] = raws[b] * vf
            idx_out[b, pl.ds(i, 1), :] = jnp.where(valids[b], bests[b], 0).reshape(1, 1)
            val_out[b, pl.ds(i, 1), :] = vf.reshape(1, 1)
        return tuple(new_ms)

    lax.fori_loop(0, MAX_DET, body, m_init)


def _nms_call(planes, raw, scores):
    return pl.pallas_call(
        _nms_body,
        out_shape=[
            jax.ShapeDtypeStruct((B, MAX_DET, 4), jnp.float32),
            jax.ShapeDtypeStruct((B, DET_PAD, 1), jnp.int32),
            jax.ShapeDtypeStruct((B, DET_PAD, 1), jnp.float32),
        ],
        scratch_shapes=[
            pltpu.VMEM((ROWS, LANES), jnp.float32),
            pltpu.VMEM((ROWS, LANES), jnp.float32),
            pltpu.VMEM((ROWS, LANES), jnp.float32),
            pltpu.VMEM((ROWS, LANES), jnp.float32),
            pltpu.VMEM((B, ROWS, LANES), jnp.float32),
            pltpu.VMEM((B, ROWS, LANES), jnp.float32),
            pltpu.VMEM((B, ROWS, LANES), jnp.float32),
            pltpu.VMEM((B, ROWS, LANES), jnp.float32),
            pltpu.VMEM((B, ROWS, LANES), jnp.float32),
        ],
    )(planes, raw, scores)


def _make_sc_gather():
    nc, ns, L = 2, 16, 16
    nw = nc * ns
    total = B * DET_PAD
    rpw = total // nw
    wpb = DET_PAD // rpw
    mesh = plsc.VectorSubcoreMesh(core_axis_name="c", subcore_axis_name="s")

    @functools.partial(
        pl.kernel,
        out_type=jax.ShapeDtypeStruct((total, 128), jnp.float32),
        mesh=mesh,
        scratch_types=[
            pltpu.VMEM((rpw,), jnp.int32),
            pltpu.VMEM((rpw, 128), jnp.float32),
            pltpu.SemaphoreType.DMA,
        ],
    )
    def sc_gather(table_hbm, idx_hbm, out_hbm, idx_v, rows_v, sem):
        wid = lax.axis_index("s") * nc + lax.axis_index("c")
        base = wid * rpw
        pltpu.sync_copy(idx_hbm.at[pl.ds(base, rpw)], idx_v)
        bi = wid // wpb
        idx_v[...] = idx_v[...] + bi * N
        pltpu.async_copy(table_hbm.at[idx_v], rows_v, sem).wait()
        pltpu.sync_copy(rows_v, out_hbm.at[pl.ds(base, rpw)])

    return sc_gather


def _mask_body(g_ref, v_ref, o_ref):
    o_ref[0] = g_ref[0, :MAX_DET, :C] * v_ref[0, :MAX_DET]


def _mask_call(g, val):
    return pl.pallas_call(
        _mask_body,
        grid=(B,),
        in_specs=[
            pl.BlockSpec((1, DET_PAD, 128), lambda b: (b, 0, 0)),
            pl.BlockSpec((1, DET_PAD, 1), lambda b: (b, 0, 0)),
        ],
        out_specs=pl.BlockSpec((1, MAX_DET, C), lambda b: (b, 0, 0)),
        out_shape=jax.ShapeDtypeStruct((B, MAX_DET, C), jnp.float32),
    )(g, val)


_sc_gather_cache = []


def _get_sc_gather():
    if not _sc_gather_cache:
        _sc_gather_cache.append(_make_sc_gather())
    return _sc_gather_cache[0]


def kernel(box_prediction, class_prediction):
    cls_pred, table, scores3 = _softmax_call(class_prediction)
    scores = scores3.reshape(B, N)
    scores_p = jnp.pad(scores, ((0, 0), (0, NPAD - N)),
                       constant_values=NEG).reshape(B, ROWS, LANES)
    planes = jnp.transpose(box_prediction, (0, 2, 1))
    planes = jnp.pad(planes, ((0, 0), (0, 0), (0, NPAD - N))).reshape(B, 4, ROWS, LANES)
    nms_box, sel_idx, sel_val = _nms_call(planes, box_prediction, scores_p)
    g = _get_sc_gather()(table.reshape(B * N, 128), sel_idx.reshape(-1))
    nms_cls = _mask_call(g.reshape(B, DET_PAD, 128), sel_val)
    return nms_box, nms_cls, cls_pred


# speedup vs baseline: 2.9507x; 1.0168x over previous
import functools

import jax
import jax.numpy as jnp
from jax import lax
from jax.experimental import pallas as pl
from jax.experimental.pallas import tpu as pltpu
from jax.experimental.pallas import tpu_sc as plsc

B, N, C = 4, 20000, 80
MAX_DET = 100
IOU_THR = 0.5
SCORE_THR = 0.5
NEG = -1e30

ROWS, LANES = 160, 128
NPAD = ROWS * LANES
NB = 10
NBLK = N // NB
DET_PAD = 128


def _softmax_body(x_ref, cls_ref, tab_ref, sc_ref):
    x = x_ref[0]
    t = x * 10.0
    sq = t * t
    m = jnp.max(sq, axis=-1, keepdims=True)
    e = jnp.exp(sq - m)
    s = jnp.sum(e, axis=-1, keepdims=True)
    p = e / s
    cls_ref[0] = p
    tab_ref[0, :, :C] = p
    tab_ref[0, :, C:] = jnp.zeros((NBLK, 128 - C), jnp.float32)
    score = 1.0 / s
    score_w = jnp.where(score >= SCORE_THR, score, NEG)
    sc_ref[0] = score_w.reshape(1, NBLK)


def _softmax_call(class_prediction):
    return pl.pallas_call(
        _softmax_body,
        grid=(B * NB,),
        in_specs=[pl.BlockSpec((1, NBLK, C), lambda i: (i // NB, i % NB, 0))],
        out_specs=[
            pl.BlockSpec((1, NBLK, C), lambda i: (i // NB, i % NB, 0)),
            pl.BlockSpec((1, NBLK, 128), lambda i: (i // NB, i % NB, 0)),
            pl.BlockSpec((1, 1, NBLK), lambda i: (i, 0, 0)),
        ],
        out_shape=[
            jax.ShapeDtypeStruct((B, N, C), jnp.float32),
            jax.ShapeDtypeStruct((B, N, 128), jnp.float32),
            jax.ShapeDtypeStruct((B * NB, 1, NBLK), jnp.float32),
        ],
    )(class_prediction)


def _nms_body(planes_ref, raw_ref, sc_ref, box_out, idx_out, val_out,
              s0, s1, s2, s3, ymin_r, xmin_r, ymax_r, xmax_r, area_r):
    s_refs = (s0, s1, s2, s3)
    idx_out[...] = jnp.zeros((B, DET_PAD, 1), jnp.int32)
    val_out[...] = jnp.zeros((B, DET_PAD, 1), jnp.float32)
    for b in range(B):
        b0 = planes_ref[b, 0]
        b1 = planes_ref[b, 1]
        b2 = planes_ref[b, 2]
        b3 = planes_ref[b, 3]
        ymin_r[b] = jnp.minimum(b0, b2)
        xmin_r[b] = jnp.minimum(b1, b3)
        ymax_r[b] = jnp.maximum(b0, b2)
        xmax_r[b] = jnp.maximum(b1, b3)
        area_r[b] = (ymax_r[b] - ymin_r[b]) * (xmax_r[b] - xmin_r[b])
        s_refs[b][...] = sc_ref[b]
    m_init = tuple(jnp.max(sc_ref[b]) for b in range(B))

    def body(i, carry):
        rr = lax.broadcasted_iota(jnp.int32, (ROWS, LANES), 0)
        cc = lax.broadcasted_iota(jnp.int32, (ROWS, LANES), 1)
        idx2d = rr * LANES + cc
        svals = [s_refs[b][...] for b in range(B)]
        valids = [carry[b] > NEG * 0.5 for b in range(B)]
        bests = [jnp.min(jnp.where(svals[b] == carry[b], idx2d, jnp.int32(2**31 - 1)))
                 for b in range(B)]
        raws = [raw_ref[b, pl.ds(bests[b], 1), :] for b in range(B)]
        new_ms = []
        sups = []
        for b in range(B):
            raw = raws[b]
            v0 = raw[0, 0]
            v1 = raw[0, 1]
            v2 = raw[0, 2]
            v3 = raw[0, 3]
            ybmin = jnp.minimum(v0, v2)
            xbmin = jnp.minimum(v1, v3)
            ybmax = jnp.maximum(v0, v2)
            xbmax = jnp.maximum(v1, v3)
            area_b = (ybmax - ybmin) * (xbmax - xbmin)
            ih = jnp.maximum(0.0, jnp.minimum(ybmax, ymax_r[b]) - jnp.maximum(ybmin, ymin_r[b]))
            iw = jnp.maximum(0.0, jnp.minimum(xbmax, xmax_r[b]) - jnp.maximum(xbmin, xmin_r[b]))
            inter = ih * iw
            union = area_b + area_r[b] - inter
            iou = jnp.where(union > 0.0, inter / union, 0.0)
            sups.append((iou > IOU_THR) | (idx2d == bests[b]))
        for b in range(B):
            new_s = jnp.where(sups[b], NEG, svals[b])
            s_refs[b][...] = new_s
            new_ms.append(jnp.max(new_s))
        for b in range(B):
            vf = jnp.where(valids[b], 1.0, 0.0)
            box_out[b, pl.ds(i, 1), :] = raws[b] * vf
            idx_out[b, pl.ds(i, 1), :] = jnp.where(valids[b], bests[b], 0).reshape(1, 1)
            val_out[b, pl.ds(i, 1), :] = vf.reshape(1, 1)
        return tuple(new_ms)

    lax.fori_loop(0, MAX_DET, body, m_init, unroll=2)


def _nms_call(planes, raw, scores):
    return pl.pallas_call(
        _nms_body,
        out_shape=[
            jax.ShapeDtypeStruct((B, MAX_DET, 4), jnp.float32),
            jax.ShapeDtypeStruct((B, DET_PAD, 1), jnp.int32),
            jax.ShapeDtypeStruct((B, DET_PAD, 1), jnp.float32),
        ],
        scratch_shapes=[
            pltpu.VMEM((ROWS, LANES), jnp.float32),
            pltpu.VMEM((ROWS, LANES), jnp.float32),
            pltpu.VMEM((ROWS, LANES), jnp.float32),
            pltpu.VMEM((ROWS, LANES), jnp.float32),
            pltpu.VMEM((B, ROWS, LANES), jnp.float32),
            pltpu.VMEM((B, ROWS, LANES), jnp.float32),
            pltpu.VMEM((B, ROWS, LANES), jnp.float32),
            pltpu.VMEM((B, ROWS, LANES), jnp.float32),
            pltpu.VMEM((B, ROWS, LANES), jnp.float32),
        ],
    )(planes, raw, scores)


def _make_sc_gather():
    nc, ns, L = 2, 16, 16
    nw = nc * ns
    total = B * DET_PAD
    rpw = total // nw
    wpb = DET_PAD // rpw
    mesh = plsc.VectorSubcoreMesh(core_axis_name="c", subcore_axis_name="s")

    @functools.partial(
        pl.kernel,
        out_type=jax.ShapeDtypeStruct((total, 128), jnp.float32),
        mesh=mesh,
        scratch_types=[
            pltpu.VMEM((rpw,), jnp.int32),
            pltpu.VMEM((rpw, 128), jnp.float32),
            pltpu.SemaphoreType.DMA,
        ],
    )
    def sc_gather(table_hbm, idx_hbm, out_hbm, idx_v, rows_v, sem):
        wid = lax.axis_index("s") * nc + lax.axis_index("c")
        base = wid * rpw
        pltpu.sync_copy(idx_hbm.at[pl.ds(base, rpw)], idx_v)
        bi = wid // wpb
        idx_v[...] = idx_v[...] + bi * N
        pltpu.async_copy(table_hbm.at[idx_v], rows_v, sem).wait()
        pltpu.sync_copy(rows_v, out_hbm.at[pl.ds(base, rpw)])

    return sc_gather


def _mask_body(g_ref, v_ref, o_ref):
    o_ref[0] = g_ref[0, :MAX_DET, :C] * v_ref[0, :MAX_DET]


def _mask_call(g, val):
    return pl.pallas_call(
        _mask_body,
        grid=(B,),
        in_specs=[
            pl.BlockSpec((1, DET_PAD, 128), lambda b: (b, 0, 0)),
            pl.BlockSpec((1, DET_PAD, 1), lambda b: (b, 0, 0)),
        ],
        out_specs=pl.BlockSpec((1, MAX_DET, C), lambda b: (b, 0, 0)),
        out_shape=jax.ShapeDtypeStruct((B, MAX_DET, C), jnp.float32),
    )(g, val)


_sc_gather_cache = []


def _get_sc_gather():
    if not _sc_gather_cache:
        _sc_gather_cache.append(_make_sc_gather())
    return _sc_gather_cache[0]


def kernel(box_prediction, class_prediction):
    cls_pred, table, scores3 = _softmax_call(class_prediction)
    scores = scores3.reshape(B, N)
    scores_p = jnp.pad(scores, ((0, 0), (0, NPAD - N)),
                       constant_values=NEG).reshape(B, ROWS, LANES)
    planes = jnp.transpose(box_prediction, (0, 2, 1))
    planes = jnp.pad(planes, ((0, 0), (0, 0), (0, NPAD - N))).reshape(B, 4, ROWS, LANES)
    nms_box, sel_idx, sel_val = _nms_call(planes, box_prediction, scores_p)
    g = _get_sc_gather()(table.reshape(B * N, 128), sel_idx.reshape(-1))
    nms_cls = _mask_call(g.reshape(B, DET_PAD, 128), sel_val)
    return nms_box, nms_cls, cls_pred


# speedup vs baseline: 2.9685x; 1.0060x over previous
import functools

import jax
import jax.numpy as jnp
from jax import lax
from jax.experimental import pallas as pl
from jax.experimental.pallas import tpu as pltpu
from jax.experimental.pallas import tpu_sc as plsc

B, N, C = 4, 20000, 80
MAX_DET = 100
IOU_THR = 0.5
SCORE_THR = 0.5
NEG = -1e30

ROWS, LANES = 160, 128
NPAD = ROWS * LANES
NB = 10
NBLK = N // NB
DET_PAD = 128


def _softmax_body(x_ref, cls_ref, tab_ref, sc_ref):
    x = x_ref[0]
    t = x * 10.0
    sq = t * t
    m = jnp.max(sq, axis=-1, keepdims=True)
    e = jnp.exp(sq - m)
    s = jnp.sum(e, axis=-1, keepdims=True)
    p = e / s
    cls_ref[0] = p
    tab_ref[0, :, :C] = p
    tab_ref[0, :, C:] = jnp.zeros((NBLK, 128 - C), jnp.float32)
    score = 1.0 / s
    score_w = jnp.where(score >= SCORE_THR, score, NEG)
    sc_ref[0] = score_w.reshape(1, NBLK)


def _softmax_call(class_prediction):
    return pl.pallas_call(
        _softmax_body,
        grid=(B * NB,),
        in_specs=[pl.BlockSpec((1, NBLK, C), lambda i: (i // NB, i % NB, 0))],
        out_specs=[
            pl.BlockSpec((1, NBLK, C), lambda i: (i // NB, i % NB, 0)),
            pl.BlockSpec((1, NBLK, 128), lambda i: (i // NB, i % NB, 0)),
            pl.BlockSpec((1, 1, NBLK), lambda i: (i, 0, 0)),
        ],
        out_shape=[
            jax.ShapeDtypeStruct((B, N, C), jnp.float32),
            jax.ShapeDtypeStruct((B, N, 128), jnp.float32),
            jax.ShapeDtypeStruct((B * NB, 1, NBLK), jnp.float32),
        ],
    )(class_prediction)


def _nms_body(planes_ref, raw_ref, sc_ref, box_out, idx_out, val_out,
              s0, s1, s2, s3, ymin_r, xmin_r, ymax_r, xmax_r, area_r):
    s_refs = (s0, s1, s2, s3)
    idx_out[...] = jnp.zeros((B, DET_PAD, 1), jnp.int32)
    val_out[...] = jnp.zeros((B, DET_PAD, 1), jnp.float32)
    for b in range(B):
        b0 = planes_ref[b, 0]
        b1 = planes_ref[b, 1]
        b2 = planes_ref[b, 2]
        b3 = planes_ref[b, 3]
        ymin_r[b] = jnp.minimum(b0, b2)
        xmin_r[b] = jnp.minimum(b1, b3)
        ymax_r[b] = jnp.maximum(b0, b2)
        xmax_r[b] = jnp.maximum(b1, b3)
        area_r[b] = (ymax_r[b] - ymin_r[b]) * (xmax_r[b] - xmin_r[b])
        s_refs[b][...] = sc_ref[b]
    m_init = tuple(jnp.max(sc_ref[b], keepdims=True) for b in range(B))

    def body(i, carry):
        rr = lax.broadcasted_iota(jnp.int32, (ROWS, LANES), 0)
        cc = lax.broadcasted_iota(jnp.int32, (ROWS, LANES), 1)
        idx2d = rr * LANES + cc
        svals = [s_refs[b][...] for b in range(B)]
        valids = [carry[b][0, 0] > NEG * 0.5 for b in range(B)]
        bests = [jnp.min(jnp.where(svals[b] == carry[b], idx2d, jnp.int32(2**31 - 1)))
                 for b in range(B)]
        raws = [raw_ref[b, pl.ds(bests[b], 1), :] for b in range(B)]
        new_ms = []
        sups = []
        for b in range(B):
            raw = raws[b]
            v0 = raw[0, 0]
            v1 = raw[0, 1]
            v2 = raw[0, 2]
            v3 = raw[0, 3]
            ybmin = jnp.minimum(v0, v2)
            xbmin = jnp.minimum(v1, v3)
            ybmax = jnp.maximum(v0, v2)
            xbmax = jnp.maximum(v1, v3)
            area_b = (ybmax - ybmin) * (xbmax - xbmin)
            ih = jnp.maximum(0.0, jnp.minimum(ybmax, ymax_r[b]) - jnp.maximum(ybmin, ymin_r[b]))
            iw = jnp.maximum(0.0, jnp.minimum(xbmax, xmax_r[b]) - jnp.maximum(xbmin, xmin_r[b]))
            inter = ih * iw
            union = area_b + area_r[b] - inter
            iou = jnp.where(union > 0.0, inter / union, 0.0)
            sups.append((iou > IOU_THR) | (idx2d == bests[b]))
        for b in range(B):
            new_s = jnp.where(sups[b], NEG, svals[b])
            s_refs[b][...] = new_s
            new_ms.append(jnp.max(new_s, keepdims=True))
        for b in range(B):
            vf = jnp.where(valids[b], 1.0, 0.0)
            box_out[b, pl.ds(i, 1), :] = raws[b] * vf
            idx_out[b, pl.ds(i, 1), :] = jnp.where(valids[b], bests[b], 0).reshape(1, 1)
            val_out[b, pl.ds(i, 1), :] = vf.reshape(1, 1)
        return tuple(new_ms)

    lax.fori_loop(0, MAX_DET, body, m_init, unroll=4)


def _nms_call(planes, raw, scores):
    return pl.pallas_call(
        _nms_body,
        out_shape=[
            jax.ShapeDtypeStruct((B, MAX_DET, 4), jnp.float32),
            jax.ShapeDtypeStruct((B, DET_PAD, 1), jnp.int32),
            jax.ShapeDtypeStruct((B, DET_PAD, 1), jnp.float32),
        ],
        scratch_shapes=[
            pltpu.VMEM((ROWS, LANES), jnp.float32),
            pltpu.VMEM((ROWS, LANES), jnp.float32),
            pltpu.VMEM((ROWS, LANES), jnp.float32),
            pltpu.VMEM((ROWS, LANES), jnp.float32),
            pltpu.VMEM((B, ROWS, LANES), jnp.float32),
            pltpu.VMEM((B, ROWS, LANES), jnp.float32),
            pltpu.VMEM((B, ROWS, LANES), jnp.float32),
            pltpu.VMEM((B, ROWS, LANES), jnp.float32),
            pltpu.VMEM((B, ROWS, LANES), jnp.float32),
        ],
    )(planes, raw, scores)


def _make_sc_gather():
    nc, ns, L = 2, 16, 16
    nw = nc * ns
    total = B * DET_PAD
    rpw = total // nw
    wpb = DET_PAD // rpw
    mesh = plsc.VectorSubcoreMesh(core_axis_name="c", subcore_axis_name="s")

    @functools.partial(
        pl.kernel,
        out_type=jax.ShapeDtypeStruct((total, 128), jnp.float32),
        mesh=mesh,
        scratch_types=[
            pltpu.VMEM((rpw,), jnp.int32),
            pltpu.VMEM((rpw, 128), jnp.float32),
            pltpu.SemaphoreType.DMA,
        ],
    )
    def sc_gather(table_hbm, idx_hbm, out_hbm, idx_v, rows_v, sem):
        wid = lax.axis_index("s") * nc + lax.axis_index("c")
        base = wid * rpw
        pltpu.sync_copy(idx_hbm.at[pl.ds(base, rpw)], idx_v)
        bi = wid // wpb
        idx_v[...] = idx_v[...] + bi * N
        pltpu.async_copy(table_hbm.at[idx_v], rows_v, sem).wait()
        pltpu.sync_copy(rows_v, out_hbm.at[pl.ds(base, rpw)])

    return sc_gather


def _mask_body(g_ref, v_ref, o_ref):
    o_ref[0] = g_ref[0, :MAX_DET, :C] * v_ref[0, :MAX_DET]


def _mask_call(g, val):
    return pl.pallas_call(
        _mask_body,
        grid=(B,),
        in_specs=[
            pl.BlockSpec((1, DET_PAD, 128), lambda b: (b, 0, 0)),
            pl.BlockSpec((1, DET_PAD, 1), lambda b: (b, 0, 0)),
        ],
        out_specs=pl.BlockSpec((1, MAX_DET, C), lambda b: (b, 0, 0)),
        out_shape=jax.ShapeDtypeStruct((B, MAX_DET, C), jnp.float32),
    )(g, val)


_sc_gather_cache = []


def _get_sc_gather():
    if not _sc_gather_cache:
        _sc_gather_cache.append(_make_sc_gather())
    return _sc_gather_cache[0]


def kernel(box_prediction, class_prediction):
    cls_pred, table, scores3 = _softmax_call(class_prediction)
    scores = scores3.reshape(B, N)
    scores_p = jnp.pad(scores, ((0, 0), (0, NPAD - N)),
                       constant_values=NEG).reshape(B, ROWS, LANES)
    planes = jnp.transpose(box_prediction, (0, 2, 1))
    planes = jnp.pad(planes, ((0, 0), (0, 0), (0, NPAD - N))).reshape(B, 4, ROWS, LANES)
    nms_box, sel_idx, sel_val = _nms_call(planes, box_prediction, scores_p)
    g = _get_sc_gather()(table.reshape(B * N, 128), sel_idx.reshape(-1))
    nms_cls = _mask_call(g.reshape(B, DET_PAD, 128), sel_val)
    return nms_box, nms_cls, cls_pred


# speedup vs baseline: 3.0141x; 1.0154x over previous
import functools

import jax
import jax.numpy as jnp
from jax import lax
from jax.experimental import pallas as pl
from jax.experimental.pallas import tpu as pltpu
from jax.experimental.pallas import tpu_sc as plsc

B, N, C = 4, 20000, 80
MAX_DET = 100
IOU_THR = 0.5
SCORE_THR = 0.5
NEG = -1e30

ROWS, LANES = 160, 128
NPAD = ROWS * LANES
NB = 5
NBLK = N // NB
DET_PAD = 128


def _softmax_body(x_ref, cls_ref, tab_ref, sc_ref):
    x = x_ref[0]
    t = x * 10.0
    sq = t * t
    m = jnp.max(sq, axis=-1, keepdims=True)
    e = jnp.exp(sq - m)
    s = jnp.sum(e, axis=-1, keepdims=True)
    p = e / s
    cls_ref[0] = p
    tab_ref[0, :, :C] = p
    tab_ref[0, :, C:] = jnp.zeros((NBLK, 128 - C), jnp.float32)
    score = 1.0 / s
    score_w = jnp.where(score >= SCORE_THR, score, NEG)
    sc_ref[0] = score_w.reshape(1, NBLK)


def _softmax_call(class_prediction):
    return pl.pallas_call(
        _softmax_body,
        grid=(B * NB,),
        in_specs=[pl.BlockSpec((1, NBLK, C), lambda i: (i // NB, i % NB, 0))],
        out_specs=[
            pl.BlockSpec((1, NBLK, C), lambda i: (i // NB, i % NB, 0)),
            pl.BlockSpec((1, NBLK, 128), lambda i: (i // NB, i % NB, 0)),
            pl.BlockSpec((1, 1, NBLK), lambda i: (i, 0, 0)),
        ],
        out_shape=[
            jax.ShapeDtypeStruct((B, N, C), jnp.float32),
            jax.ShapeDtypeStruct((B, N, 128), jnp.float32),
            jax.ShapeDtypeStruct((B * NB, 1, NBLK), jnp.float32),
        ],
    )(class_prediction)


def _nms_body(planes_ref, raw_ref, sc_ref, box_out, idx_out, val_out,
              s0, s1, s2, s3, ymin_r, xmin_r, ymax_r, xmax_r, area_r):
    s_refs = (s0, s1, s2, s3)
    idx_out[...] = jnp.zeros((B, DET_PAD, 1), jnp.int32)
    val_out[...] = jnp.zeros((B, DET_PAD, 1), jnp.float32)
    for b in range(B):
        b0 = planes_ref[b, 0]
        b1 = planes_ref[b, 1]
        b2 = planes_ref[b, 2]
        b3 = planes_ref[b, 3]
        ymin_r[b] = jnp.minimum(b0, b2)
        xmin_r[b] = jnp.minimum(b1, b3)
        ymax_r[b] = jnp.maximum(b0, b2)
        xmax_r[b] = jnp.maximum(b1, b3)
        area_r[b] = (ymax_r[b] - ymin_r[b]) * (xmax_r[b] - xmin_r[b])
        s_refs[b][...] = sc_ref[b]
    m_init = tuple(jnp.max(sc_ref[b], keepdims=True) for b in range(B))

    def body(i, carry):
        rr = lax.broadcasted_iota(jnp.int32, (ROWS, LANES), 0)
        cc = lax.broadcasted_iota(jnp.int32, (ROWS, LANES), 1)
        idx2d = rr * LANES + cc
        svals = [s_refs[b][...] for b in range(B)]
        valids = [carry[b][0, 0] > NEG * 0.5 for b in range(B)]
        bests = [jnp.min(jnp.where(svals[b] == carry[b], idx2d, jnp.int32(2**31 - 1)))
                 for b in range(B)]
        raws = [raw_ref[b, pl.ds(bests[b], 1), :] for b in range(B)]
        new_ms = []
        sups = []
        for b in range(B):
            raw = raws[b]
            v0 = raw[0, 0]
            v1 = raw[0, 1]
            v2 = raw[0, 2]
            v3 = raw[0, 3]
            ybmin = jnp.minimum(v0, v2)
            xbmin = jnp.minimum(v1, v3)
            ybmax = jnp.maximum(v0, v2)
            xbmax = jnp.maximum(v1, v3)
            area_b = (ybmax - ybmin) * (xbmax - xbmin)
            ih = jnp.maximum(0.0, jnp.minimum(ybmax, ymax_r[b]) - jnp.maximum(ybmin, ymin_r[b]))
            iw = jnp.maximum(0.0, jnp.minimum(xbmax, xmax_r[b]) - jnp.maximum(xbmin, xmin_r[b]))
            inter = ih * iw
            union = area_b + area_r[b] - inter
            iou = jnp.where(union > 0.0, inter / union, 0.0)
            sups.append((iou > IOU_THR) | (idx2d == bests[b]))
        for b in range(B):
            new_s = jnp.where(sups[b], NEG, svals[b])
            s_refs[b][...] = new_s
            new_ms.append(jnp.max(new_s, keepdims=True))
        for b in range(B):
            vf = jnp.where(valids[b], 1.0, 0.0)
            box_out[b, pl.ds(i, 1), :] = raws[b] * vf
            idx_out[b, pl.ds(i, 1), :] = jnp.where(valids[b], bests[b], 0).reshape(1, 1)
            val_out[b, pl.ds(i, 1), :] = vf.reshape(1, 1)
        return tuple(new_ms)

    lax.fori_loop(0, MAX_DET, body, m_init, unroll=4)


def _nms_call(planes, raw, scores):
    return pl.pallas_call(
        _nms_body,
        out_shape=[
            jax.ShapeDtypeStruct((B, MAX_DET, 4), jnp.float32),
            jax.ShapeDtypeStruct((B, DET_PAD, 1), jnp.int32),
            jax.ShapeDtypeStruct((B, DET_PAD, 1), jnp.float32),
        ],
        scratch_shapes=[
            pltpu.VMEM((ROWS, LANES), jnp.float32),
            pltpu.VMEM((ROWS, LANES), jnp.float32),
            pltpu.VMEM((ROWS, LANES), jnp.float32),
            pltpu.VMEM((ROWS, LANES), jnp.float32),
            pltpu.VMEM((B, ROWS, LANES), jnp.float32),
            pltpu.VMEM((B, ROWS, LANES), jnp.float32),
            pltpu.VMEM((B, ROWS, LANES), jnp.float32),
            pltpu.VMEM((B, ROWS, LANES), jnp.float32),
            pltpu.VMEM((B, ROWS, LANES), jnp.float32),
        ],
    )(planes, raw, scores)


def _make_sc_gather():
    nc, ns, L = 2, 16, 16
    nw = nc * ns
    total = B * DET_PAD
    rpw = total // nw
    wpb = DET_PAD // rpw
    mesh = plsc.VectorSubcoreMesh(core_axis_name="c", subcore_axis_name="s")

    @functools.partial(
        pl.kernel,
        out_type=jax.ShapeDtypeStruct((total, 128), jnp.float32),
        mesh=mesh,
        scratch_types=[
            pltpu.VMEM((rpw,), jnp.int32),
            pltpu.VMEM((rpw, 128), jnp.float32),
            pltpu.SemaphoreType.DMA,
        ],
    )
    def sc_gather(table_hbm, idx_hbm, out_hbm, idx_v, rows_v, sem):
        wid = lax.axis_index("s") * nc + lax.axis_index("c")
        base = wid * rpw
        pltpu.sync_copy(idx_hbm.at[pl.ds(base, rpw)], idx_v)
        bi = wid // wpb
        idx_v[...] = idx_v[...] + bi * N
        pltpu.async_copy(table_hbm.at[idx_v], rows_v, sem).wait()
        pltpu.sync_copy(rows_v, out_hbm.at[pl.ds(base, rpw)])

    return sc_gather


def _mask_body(g_ref, v_ref, o_ref):
    o_ref[0] = g_ref[0, :MAX_DET, :C] * v_ref[0, :MAX_DET]


def _mask_call(g, val):
    return pl.pallas_call(
        _mask_body,
        grid=(B,),
        in_specs=[
            pl.BlockSpec((1, DET_PAD, 128), lambda b: (b, 0, 0)),
            pl.BlockSpec((1, DET_PAD, 1), lambda b: (b, 0, 0)),
        ],
        out_specs=pl.BlockSpec((1, MAX_DET, C), lambda b: (b, 0, 0)),
        out_shape=jax.ShapeDtypeStruct((B, MAX_DET, C), jnp.float32),
    )(g, val)


_sc_gather_cache = []


def _get_sc_gather():
    if not _sc_gather_cache:
        _sc_gather_cache.append(_make_sc_gather())
    return _sc_gather_cache[0]


def kernel(box_prediction, class_prediction):
    cls_pred, table, scores3 = _softmax_call(class_prediction)
    scores = scores3.reshape(B, N)
    scores_p = jnp.pad(scores, ((0, 0), (0, NPAD - N)),
                       constant_values=NEG).reshape(B, ROWS, LANES)
    planes = jnp.transpose(box_prediction, (0, 2, 1))
    planes = jnp.pad(planes, ((0, 0), (0, 0), (0, NPAD - N))).reshape(B, 4, ROWS, LANES)
    nms_box, sel_idx, sel_val = _nms_call(planes, box_prediction, scores_p)
    g = _get_sc_gather()(table.reshape(B * N, 128), sel_idx.reshape(-1))
    nms_cls = _mask_call(g.reshape(B, DET_PAD, 128), sel_val)
    return nms_box, nms_cls, cls_pred


# speedup vs baseline: 3.0485x; 1.0114x over previous
import functools

import jax
import jax.numpy as jnp
from jax import lax
from jax.experimental import pallas as pl
from jax.experimental.pallas import tpu as pltpu
from jax.experimental.pallas import tpu_sc as plsc

B, N, C = 4, 20000, 80
MAX_DET = 100
IOU_THR = 0.5
SCORE_THR = 0.5
NEG = -1e30

ROWS, LANES = 160, 128
NPAD = ROWS * LANES
NB = 2
NBLK = N // NB
DET_PAD = 128


def _softmax_body(x_ref, cls_ref, tab_ref, sc_ref):
    x = x_ref[0]
    t = x * 10.0
    sq = t * t
    m = jnp.max(sq, axis=-1, keepdims=True)
    e = jnp.exp(sq - m)
    s = jnp.sum(e, axis=-1, keepdims=True)
    p = e / s
    cls_ref[0] = p
    tab_ref[0, :, :C] = p
    tab_ref[0, :, C:] = jnp.zeros((NBLK, 128 - C), jnp.float32)
    score = 1.0 / s
    score_w = jnp.where(score >= SCORE_THR, score, NEG)
    sc_ref[0] = score_w.reshape(1, NBLK)


def _softmax_call(class_prediction):
    return pl.pallas_call(
        _softmax_body,
        grid=(B * NB,),
        in_specs=[pl.BlockSpec((1, NBLK, C), lambda i: (i // NB, i % NB, 0))],
        out_specs=[
            pl.BlockSpec((1, NBLK, C), lambda i: (i // NB, i % NB, 0)),
            pl.BlockSpec((1, NBLK, 128), lambda i: (i // NB, i % NB, 0)),
            pl.BlockSpec((1, 1, NBLK), lambda i: (i, 0, 0)),
        ],
        out_shape=[
            jax.ShapeDtypeStruct((B, N, C), jnp.float32),
            jax.ShapeDtypeStruct((B, N, 128), jnp.float32),
            jax.ShapeDtypeStruct((B * NB, 1, NBLK), jnp.float32),
        ],
    )(class_prediction)


def _nms_body(planes_ref, raw_ref, sc_ref, box_out, idx_out, val_out,
              s0, s1, s2, s3, ymin_r, xmin_r, ymax_r, xmax_r, area_r):
    s_refs = (s0, s1, s2, s3)
    idx_out[...] = jnp.zeros((B, DET_PAD, 1), jnp.int32)
    val_out[...] = jnp.zeros((B, DET_PAD, 1), jnp.float32)
    for b in range(B):
        b0 = planes_ref[b, 0]
        b1 = planes_ref[b, 1]
        b2 = planes_ref[b, 2]
        b3 = planes_ref[b, 3]
        ymin_r[b] = jnp.minimum(b0, b2)
        xmin_r[b] = jnp.minimum(b1, b3)
        ymax_r[b] = jnp.maximum(b0, b2)
        xmax_r[b] = jnp.maximum(b1, b3)
        area_r[b] = (ymax_r[b] - ymin_r[b]) * (xmax_r[b] - xmin_r[b])
        s_refs[b][...] = sc_ref[b]
    m_init = tuple(jnp.max(sc_ref[b], keepdims=True) for b in range(B))

    def body(i, carry):
        rr = lax.broadcasted_iota(jnp.int32, (ROWS, LANES), 0)
        cc = lax.broadcasted_iota(jnp.int32, (ROWS, LANES), 1)
        idx2d = rr * LANES + cc
        svals = [s_refs[b][...] for b in range(B)]
        valids = [carry[b][0, 0] > NEG * 0.5 for b in range(B)]
        bests = [jnp.min(jnp.where(svals[b] == carry[b], idx2d, jnp.int32(2**31 - 1)))
                 for b in range(B)]
        raws = [raw_ref[b, pl.ds(bests[b], 1), :] for b in range(B)]
        new_ms = []
        sups = []
        for b in range(B):
            raw = raws[b]
            v0 = raw[0, 0]
            v1 = raw[0, 1]
            v2 = raw[0, 2]
            v3 = raw[0, 3]
            ybmin = jnp.minimum(v0, v2)
            xbmin = jnp.minimum(v1, v3)
            ybmax = jnp.maximum(v0, v2)
            xbmax = jnp.maximum(v1, v3)
            area_b = (ybmax - ybmin) * (xbmax - xbmin)
            ih = jnp.maximum(0.0, jnp.minimum(ybmax, ymax_r[b]) - jnp.maximum(ybmin, ymin_r[b]))
            iw = jnp.maximum(0.0, jnp.minimum(xbmax, xmax_r[b]) - jnp.maximum(xbmin, xmin_r[b]))
            inter = ih * iw
            union = area_b + area_r[b] - inter
            iou = jnp.where(union > 0.0, inter / union, 0.0)
            sups.append((iou > IOU_THR) | (idx2d == bests[b]))
        for b in range(B):
            new_s = jnp.where(sups[b], NEG, svals[b])
            s_refs[b][...] = new_s
            new_ms.append(jnp.max(new_s, keepdims=True))
        for b in range(B):
            vf = jnp.where(valids[b], 1.0, 0.0)
            box_out[b, pl.ds(i, 1), :] = raws[b] * vf
            idx_out[b, pl.ds(i, 1), :] = jnp.where(valids[b], bests[b], 0).reshape(1, 1)
            val_out[b, pl.ds(i, 1), :] = vf.reshape(1, 1)
        return tuple(new_ms)

    lax.fori_loop(0, MAX_DET, body, m_init, unroll=4)


def _nms_call(planes, raw, scores):
    return pl.pallas_call(
        _nms_body,
        out_shape=[
            jax.ShapeDtypeStruct((B, MAX_DET, 4), jnp.float32),
            jax.ShapeDtypeStruct((B, DET_PAD, 1), jnp.int32),
            jax.ShapeDtypeStruct((B, DET_PAD, 1), jnp.float32),
        ],
        scratch_shapes=[
            pltpu.VMEM((ROWS, LANES), jnp.float32),
            pltpu.VMEM((ROWS, LANES), jnp.float32),
            pltpu.VMEM((ROWS, LANES), jnp.float32),
            pltpu.VMEM((ROWS, LANES), jnp.float32),
            pltpu.VMEM((B, ROWS, LANES), jnp.float32),
            pltpu.VMEM((B, ROWS, LANES), jnp.float32),
            pltpu.VMEM((B, ROWS, LANES), jnp.float32),
            pltpu.VMEM((B, ROWS, LANES), jnp.float32),
            pltpu.VMEM((B, ROWS, LANES), jnp.float32),
        ],
    )(planes, raw, scores)


def _make_sc_gather():
    nc, ns, L = 2, 16, 16
    nw = nc * ns
    total = B * DET_PAD
    rpw = total // nw
    wpb = DET_PAD // rpw
    mesh = plsc.VectorSubcoreMesh(core_axis_name="c", subcore_axis_name="s")

    @functools.partial(
        pl.kernel,
        out_type=jax.ShapeDtypeStruct((total, 128), jnp.float32),
        mesh=mesh,
        scratch_types=[
            pltpu.VMEM((rpw,), jnp.int32),
            pltpu.VMEM((rpw, 128), jnp.float32),
            pltpu.SemaphoreType.DMA,
        ],
    )
    def sc_gather(table_hbm, idx_hbm, out_hbm, idx_v, rows_v, sem):
        wid = lax.axis_index("s") * nc + lax.axis_index("c")
        base = wid * rpw
        pltpu.sync_copy(idx_hbm.at[pl.ds(base, rpw)], idx_v)
        bi = wid // wpb
        idx_v[...] = idx_v[...] + bi * N
        pltpu.async_copy(table_hbm.at[idx_v], rows_v, sem).wait()
        pltpu.sync_copy(rows_v, out_hbm.at[pl.ds(base, rpw)])

    return sc_gather


def _mask_body(g_ref, v_ref, o_ref):
    o_ref[0] = g_ref[0, :MAX_DET, :C] * v_ref[0, :MAX_DET]


def _mask_call(g, val):
    return pl.pallas_call(
        _mask_body,
        grid=(B,),
        in_specs=[
            pl.BlockSpec((1, DET_PAD, 128), lambda b: (b, 0, 0)),
            pl.BlockSpec((1, DET_PAD, 1), lambda b: (b, 0, 0)),
        ],
        out_specs=pl.BlockSpec((1, MAX_DET, C), lambda b: (b, 0, 0)),
        out_shape=jax.ShapeDtypeStruct((B, MAX_DET, C), jnp.float32),
    )(g, val)


_sc_gather_cache = []


def _get_sc_gather():
    if not _sc_gather_cache:
        _sc_gather_cache.append(_make_sc_gather())
    return _sc_gather_cache[0]


def kernel(box_prediction, class_prediction):
    cls_pred, table, scores3 = _softmax_call(class_prediction)
    scores = scores3.reshape(B, N)
    scores_p = jnp.pad(scores, ((0, 0), (0, NPAD - N)),
                       constant_values=NEG).reshape(B, ROWS, LANES)
    planes = jnp.transpose(box_prediction, (0, 2, 1))
    planes = jnp.pad(planes, ((0, 0), (0, 0), (0, NPAD - N))).reshape(B, 4, ROWS, LANES)
    nms_box, sel_idx, sel_val = _nms_call(planes, box_prediction, scores_p)
    g = _get_sc_gather()(table.reshape(B * N, 128), sel_idx.reshape(-1))
    nms_cls = _mask_call(g.reshape(B, DET_PAD, 128), sel_val)
    return nms_box, nms_cls, cls_pred
